# baseline (speedup 1.0000x reference)
import functools

import jax
import jax.numpy as jnp
from jax import lax
from jax.experimental import pallas as pl
from jax.experimental.pallas import tpu as pltpu
from jax.experimental.pallas import tpu_sc as plsc

N = 10000
E = 320000
G = 64
H = 128
NC, NS, L = 2, 16, 16
NW = NC * NS
EC = E // NW
NP = 10240
NCHUNK = NP // NS
CHUNK = 80
NCH = EC // CHUNK
RB = 640
EB = 8000
NEG = -3.0e38

_MESH = plsc.VectorSubcoreMesh(
    core_axis_name="c", subcore_axis_name="s", num_cores=NC, num_subcores=NS)
_SC_PARAMS = pltpu.CompilerParams(needs_layout_passes=False)


def _bf(t):
    return t.astype(jnp.bfloat16)


def _t0s_body(a_ref, o_ref, we_ref, be_ref, weo_ref, beo_ref,
              wl1_ref, bl1_ref, ea1_ref):
    ee = jnp.concatenate(
        [a_ref[...] * we_ref[...] + be_ref[...],
         o_ref[...] * weo_ref[...] + beo_ref[...]], axis=1)
    ea1_ref[...] = (jnp.dot(_bf(ee), wl1_ref[...],
                            preferred_element_type=jnp.float32) + bl1_ref[...])


def _t0h_body(a_ref, o_ref, we_ref, be_ref, weo_ref, beo_ref,
              wl_ref, bl_ref, ea_ref):
    ee = jnp.concatenate(
        [a_ref[...] * we_ref[...] + be_ref[...],
         o_ref[...] * weo_ref[...] + beo_ref[...]], axis=1)
    ea_ref[...] = (jnp.dot(_bf(ee), wl_ref[...],
                           preferred_element_type=jnp.float32) + bl_ref[...])


_EE_SPECS = [
    pl.BlockSpec((EB, 1), lambda i: (i, 0)),
    pl.BlockSpec((EB, 1), lambda i: (i, 0)),
    pl.BlockSpec((1, H // 2), lambda i: (0, 0)),
    pl.BlockSpec((1, H // 2), lambda i: (0, 0)),
    pl.BlockSpec((1, H // 2), lambda i: (0, 0)),
    pl.BlockSpec((1, H // 2), lambda i: (0, 0)),
]


def _t0s(a2, o2, We, be, Weo, beo, Wl1, bl1):
    return pl.pallas_call(
        _t0s_body,
        grid=(E // EB,),
        in_specs=_EE_SPECS + [
            pl.BlockSpec((H, 1), lambda i: (0, 0)),
            pl.BlockSpec((1, 1), lambda i: (0, 0)),
        ],
        out_specs=pl.BlockSpec((EB, 1), lambda i: (i, 0)),
        out_shape=jax.ShapeDtypeStruct((E, 1), jnp.float32),
    )(a2, o2, We, be.reshape(1, -1), Weo, beo.reshape(1, -1),
      _bf(Wl1), bl1.reshape(1, 1))


def _t0h(a2, o2, We, be, Weo, beo, Wl, bl):
    return pl.pallas_call(
        _t0h_body,
        grid=(E // EB,),
        in_specs=_EE_SPECS + [
            pl.BlockSpec((H, H), lambda i: (0, 0)),
            pl.BlockSpec((1, H), lambda i: (0, 0)),
        ],
        out_specs=pl.BlockSpec((EB, H), lambda i: (i, 0)),
        out_shape=jax.ShapeDtypeStruct((E, H), jnp.float32),
    )(a2, o2, We, be.reshape(1, -1), Weo, beo.reshape(1, -1),
      _bf(Wl), bl.reshape(1, H))


@functools.partial(
    pl.kernel,
    out_type=jax.ShapeDtypeStruct((NC, NP), jnp.float32),
    mesh=_MESH,
    compiler_params=_SC_PARAMS,
    scratch_types=dict(
        xs_v=pltpu.VMEM((NP,), jnp.float32),
        src_v=pltpu.VMEM((EC,), jnp.int32),
        dst_v=pltpu.VMEM((EC,), jnp.int32),
        ea_v=pltpu.VMEM((EC,), jnp.float32),
        acc_v=pltpu.VMEM((NP,), jnp.float32),
        red_v=pltpu.VMEM((NS, NCHUNK), jnp.float32),
        out_v=pltpu.VMEM((NCHUNK,), jnp.float32),
        shared=pltpu.VMEM_SHARED((NS, NP), jnp.float32),
    ),
)
def _k1(xs_hbm, ei_hbm, ea_hbm, out_hbm,
        xs_v, src_v, dst_v, ea_v, acc_v, red_v, out_v, shared):
    cid = lax.axis_index("c")
    sid = lax.axis_index("s")
    wid = sid * NC + cid
    ebase = wid * EC

    pltpu.sync_copy(xs_hbm, xs_v)
    pltpu.sync_copy(ei_hbm.at[pl.ds(ebase, EC)], src_v)
    pltpu.sync_copy(ei_hbm.at[pl.ds(E + ebase, EC)], dst_v)
    pltpu.sync_copy(ea_hbm.at[pl.ds(ebase, EC)], ea_v)

    zeros = jnp.zeros((L,), jnp.float32)

    def _zero(i, _):
        acc_v[pl.ds(i * L, L)] = zeros
        return 0
    lax.fori_loop(0, NP // L, _zero, 0)

    def _edge_group(g, _):
        idx = src_v[pl.ds(g * L, L)]
        xg = plsc.load_gather(xs_v, [idx])
        m = jnp.maximum(xg + ea_v[pl.ds(g * L, L)], 0.0)
        d = dst_v[pl.ds(g * L, L)]
        plsc.addupdate_scatter(acc_v, [d], m)
        return 0
    lax.fori_loop(0, EC // L, _edge_group, 0)

    pltpu.sync_copy(acc_v, shared.at[sid])
    plsc.subcore_barrier()
    pltpu.sync_copy(shared.at[:, pl.ds(sid * NCHUNK, NCHUNK)], red_v)

    def _red(j, _):
        s = red_v[0, pl.ds(j * L, L)]
        for rrow in range(1, NS):
            s = s + red_v[rrow, pl.ds(j * L, L)]
        out_v[pl.ds(j * L, L)] = s
        return 0
    lax.fori_loop(0, NCHUNK // L, _red, 0)
    pltpu.sync_copy(out_v, out_hbm.at[cid, pl.ds(sid * NCHUNK, NCHUNK)])


@functools.partial(
    pl.kernel,
    out_type=jax.ShapeDtypeStruct((NC, NP, H), jnp.float32),
    mesh=_MESH,
    compiler_params=_SC_PARAMS,
    scratch_types=dict(
        src_v=pltpu.VMEM((EC,), jnp.int32),
        rows0=pltpu.VMEM((CHUNK, H), jnp.float32),
        rows1=pltpu.VMEM((CHUNK, H), jnp.float32),
        ea_v=pltpu.VMEM((CHUNK, H), jnp.float32),
        didx0=pltpu.VMEM((CHUNK,), jnp.int32),
        didx1=pltpu.VMEM((CHUNK,), jnp.int32),
        acc_s=pltpu.VMEM_SHARED((NP, H), jnp.float32),
        g_sem0=pltpu.SemaphoreType.DMA,
        g_sem1=pltpu.SemaphoreType.DMA,
        ea_sem=pltpu.SemaphoreType.DMA,
        d_sem0=pltpu.SemaphoreType.DMA,
        d_sem1=pltpu.SemaphoreType.DMA,
    ),
)
def _edge_pass(table_hbm, ei_hbm, ea_hbm, out_hbm,
               src_v, rows0, rows1, ea_v, didx0, didx1, acc_s,
               g_sem0, g_sem1, ea_sem, d_sem0, d_sem1):
    cid = lax.axis_index("c")
    sid = lax.axis_index("s")
    wid = sid * NC + cid
    ebase = wid * EC
    rbase = sid * NCHUNK
    rows = (rows0, rows1)
    didx = (didx0, didx1)
    g_sem = (g_sem0, g_sem1)
    d_sem = (d_sem0, d_sem1)

    pltpu.sync_copy(ei_hbm.at[pl.ds(ebase, EC)], src_v)

    zeros = jnp.zeros((L,), jnp.float32)

    def _zrow(i, _):
        for blk in range(H // L):
            rows0[i, pl.ds(blk * L, L)] = zeros
        return 0
    lax.fori_loop(0, CHUNK, _zrow, 0)
    for piece in range(NCHUNK // CHUNK):
        pltpu.sync_copy(rows0, acc_s.at[pl.ds(rbase + piece * CHUNK, CHUNK)])
    plsc.subcore_barrier()

    def _issue_gather(c, b):
        return pltpu.async_copy(
            table_hbm.at[src_v.at[pl.ds(c * CHUNK, CHUNK)]], rows[b], g_sem[b])

    def _issue_didx(c, b):
        return pltpu.async_copy(
            ei_hbm.at[pl.ds(E + ebase + c * CHUNK, CHUNK)], didx[b], d_sem[b])

    def _issue_ea(c):
        return pltpu.async_copy(
            ea_hbm.at[pl.ds(ebase + c * CHUNK, CHUNK)], ea_v, ea_sem)

    def _relu_chunk(b):
        def body(r, _):
            for blk in range(H // L):
                sl = pl.ds(blk * L, L)
                rows[b][r, sl] = jnp.maximum(rows[b][r, sl] + ea_v[r, sl], 0.0)
            return 0
        lax.fori_loop(0, CHUNK, body, 0)

    def _step(c, b):
        pltpu.make_async_copy(
            table_hbm.at[src_v.at[pl.ds(0, CHUNK)]], rows[b], g_sem[b]).wait()
        pltpu.make_async_copy(
            ea_hbm.at[pl.ds(0, CHUNK)], ea_v, ea_sem).wait()
        pltpu.make_async_copy(
            ei_hbm.at[pl.ds(0, CHUNK)], didx[b], d_sem[b]).wait()
        _relu_chunk(b)

        @pl.when(c + 1 < NCH)
        def _():
            _issue_ea(c + 1)
        pltpu.sync_copy(rows[b], acc_s.at[didx[b]], add=True)

        @pl.when(c + 2 < NCH)
        def _():
            _issue_gather(c + 2, b)
            _issue_didx(c + 2, b)

    _issue_gather(0, 0)
    _issue_didx(0, 0)
    _issue_gather(1, 1)
    _issue_didx(1, 1)
    _issue_ea(0)

    def _pair(i, _):
        _step(2 * i, 0)
        _step(2 * i + 1, 1)
        return 0
    lax.fori_loop(0, NCH // 2, _pair, 0)
    if NCH % 2:
        _step(NCH - 1, 0)
    plsc.subcore_barrier()

    for piece in range(NCHUNK // CHUNK):
        pbase = rbase + piece * CHUNK
        pltpu.sync_copy(acc_s.at[pl.ds(pbase, CHUNK)], rows0)
        pltpu.sync_copy(rows0, out_hbm.at[cid, pl.ds(pbase, CHUNK)])


def _k3a_body(x_ref, p0_ref, p1_ref, w_ref, b_ref, o_ref):
    g1 = x_ref[...] + p0_ref[...] + p1_ref[...]
    o_ref[...] = g1 * w_ref[...] + b_ref[...]


def _k3a(xs2, p02, p12, w1r, bn1):
    cspec = pl.BlockSpec((RB, 1), lambda i: (i, 0))
    return pl.pallas_call(
        _k3a_body,
        grid=(NP // RB,),
        in_specs=[cspec, cspec, cspec,
                  pl.BlockSpec((1, H), lambda i: (0, 0)),
                  pl.BlockSpec((1, H), lambda i: (0, 0))],
        out_specs=pl.BlockSpec((RB, H), lambda i: (i, 0)),
        out_shape=jax.ShapeDtypeStruct((NP, H), jnp.float32),
    )(xs2, p02, p12, w1r, bn1.reshape(1, H))


def _k3_body(h1_ref, a_ref, w_ref, b_ref, o_ref):
    t = _bf(h1_ref[...] + a_ref[0] + a_ref[1])
    o_ref[...] = jnp.dot(t, w_ref[...], preferred_element_type=jnp.float32) + b_ref[...]


def _k3(h1, A, Wn, bn):
    return pl.pallas_call(
        _k3_body,
        grid=(NP // RB,),
        in_specs=[
            pl.BlockSpec((RB, H), lambda i: (i, 0)),
            pl.BlockSpec((NC, RB, H), lambda i: (0, i, 0)),
            pl.BlockSpec((H, H), lambda i: (0, 0)),
            pl.BlockSpec((1, H), lambda i: (0, 0)),
        ],
        out_specs=pl.BlockSpec((RB, H), lambda i: (i, 0)),
        out_shape=jax.ShapeDtypeStruct((NP, H), jnp.float32),
    )(h1, A, _bf(Wn), bn.reshape(1, H))


def _k5_body(h2_ref, b_ref, w3_ref, b3_ref, bid_ref, wf1_ref, bf1_ref,
             wf2_ref, bf2_ref, o_ref, pool_ref):
    i = pl.program_id(0)

    @pl.when(i == 0)
    def _():
        pool_ref[...] = jnp.full((G, H), NEG, jnp.float32)

    t = _bf(h2_ref[...] + b_ref[0] + b_ref[1])
    h3 = jnp.dot(t, w3_ref[...], preferred_element_type=jnp.float32) + b3_ref[...]
    bid = bid_ref[...]
    bmin = bid_ref[0, 0]
    bmax = bid_ref[RB - 1, 0]
    for g in range(G):
        @pl.when((g >= bmin) & (g <= bmax))
        def _():
            masked = jnp.where(bid == g, h3, NEG)
            red = jnp.max(masked, axis=0)
            pool_ref[g, :] = jnp.maximum(pool_ref[g, :], red)

    @pl.when(i == NP // RB - 1)
    def _():
        pooled = pool_ref[...]
        z = jax.nn.relu(
            jnp.dot(_bf(pooled), wf1_ref[...], preferred_element_type=jnp.float32)
            + bf1_ref[...])
        o_ref[...] = (jnp.dot(_bf(z), wf2_ref[...], preferred_element_type=jnp.float32)
                      + bf2_ref[...])


def _k5(h2, B, Wn3, bn3, bid, Wf1, bf1, Wf2, bf2):
    return pl.pallas_call(
        _k5_body,
        grid=(NP // RB,),
        in_specs=[
            pl.BlockSpec((RB, H), lambda i: (i, 0)),
            pl.BlockSpec((NC, RB, H), lambda i: (0, i, 0)),
            pl.BlockSpec((H, H), lambda i: (0, 0)),
            pl.BlockSpec((1, H), lambda i: (0, 0)),
            pl.BlockSpec((RB, 1), lambda i: (i, 0)),
            pl.BlockSpec((H, H), lambda i: (0, 0)),
            pl.BlockSpec((1, H), lambda i: (0, 0)),
            pl.BlockSpec((H, 1), lambda i: (0, 0)),
            pl.BlockSpec((1, 1), lambda i: (0, 0)),
        ],
        out_specs=pl.BlockSpec((G, 1), lambda i: (0, 0)),
        out_shape=jax.ShapeDtypeStruct((G, 1), jnp.float32),
        scratch_shapes=[pltpu.VMEM((G, H), jnp.float32)],
    )(h2, B, _bf(Wn3), bn3.reshape(1, H), bid, _bf(Wf1), bf1.reshape(1, H),
      _bf(Wf2), bf2.reshape(1, 1))


def kernel(x, edge_index, edge_attr, batch, opt_edge, We, be, Weo, beo, Wl1, bl1, Wn1, bn1, Wl2, bl2, Wn2, bn2, Wl3, bl3, Wn3, bn3, Wf1, bf1, Wf2, bf2):
    xs = jnp.pad(x[:, 0], (0, NP - N))
    eif = edge_index.reshape(2 * E)

    ea1 = _t0s(edge_attr, opt_edge, We, be, Weo, beo, Wl1, bl1)
    part1 = _k1(xs, eif, ea1[:, 0])
    ea2 = _t0h(edge_attr, opt_edge, We, be, Weo, beo, Wl2, bl2)

    w1r = Wn1[0].reshape(1, H)
    h1 = _k3a(xs.reshape(NP, 1), part1[0].reshape(NP, 1),
              part1[1].reshape(NP, 1), w1r, bn1)

    A = _edge_pass(h1, eif, ea2)
    ea3 = _t0h(edge_attr, opt_edge, We, be, Weo, beo, Wl3, bl3)
    h2 = _k3(h1, A, Wn2, bn2)

    B = _edge_pass(h2, eif, ea3)

    bid = jnp.pad(batch, (0, NP - N), constant_values=G).reshape(NP, 1)
    return _k5(h2, B, Wn3, bn3, bid, Wf1, bf1, Wf2, bf2)


# speedup vs baseline: 1.1007x; 1.1007x over previous
import functools

import jax
import jax.numpy as jnp
from jax import lax
from jax.experimental import pallas as pl
from jax.experimental.pallas import tpu as pltpu
from jax.experimental.pallas import tpu_sc as plsc

N = 10000
E = 320000
G = 64
H = 128
NC, NS, L = 2, 16, 16
NW = NC * NS
EC = E // NW
NP = 10240
NCHUNK = NP // NS
CHUNK = 80
NCH = EC // CHUNK
RB = 640
EB = 8000
NEG = -3.0e38

_MESH = plsc.VectorSubcoreMesh(
    core_axis_name="c", subcore_axis_name="s", num_cores=NC, num_subcores=NS)
_SC_PARAMS = pltpu.CompilerParams(needs_layout_passes=False)


def _bf(t):
    return t.astype(jnp.bfloat16)


def _t0a_body(a_ref, o_ref, we_ref, be_ref, weo_ref, beo_ref,
              wl1_ref, bl1_ref, wl2_ref, bl2_ref, ea1_ref, ea2_ref):
    ee = jnp.concatenate(
        [a_ref[...] * we_ref[...] + be_ref[...],
         o_ref[...] * weo_ref[...] + beo_ref[...]], axis=1)
    eeb = _bf(ee)
    ea1_ref[...] = (jnp.dot(eeb, wl1_ref[...],
                            preferred_element_type=jnp.float32) + bl1_ref[...])
    ea2_ref[...] = (jnp.dot(eeb, wl2_ref[...],
                            preferred_element_type=jnp.float32) + bl2_ref[...])


def _t0h_body(a_ref, o_ref, we_ref, be_ref, weo_ref, beo_ref,
              wl_ref, bl_ref, ea_ref):
    ee = jnp.concatenate(
        [a_ref[...] * we_ref[...] + be_ref[...],
         o_ref[...] * weo_ref[...] + beo_ref[...]], axis=1)
    ea_ref[...] = (jnp.dot(_bf(ee), wl_ref[...],
                           preferred_element_type=jnp.float32) + bl_ref[...])


_EE_SPECS = [
    pl.BlockSpec((EB, 1), lambda i: (i, 0)),
    pl.BlockSpec((EB, 1), lambda i: (i, 0)),
    pl.BlockSpec((1, H // 2), lambda i: (0, 0)),
    pl.BlockSpec((1, H // 2), lambda i: (0, 0)),
    pl.BlockSpec((1, H // 2), lambda i: (0, 0)),
    pl.BlockSpec((1, H // 2), lambda i: (0, 0)),
]


def _t0a(a2, o2, We, be, Weo, beo, Wl1, bl1, Wl2, bl2):
    return pl.pallas_call(
        _t0a_body,
        grid=(E // EB,),
        in_specs=_EE_SPECS + [
            pl.BlockSpec((H, 1), lambda i: (0, 0)),
            pl.BlockSpec((1, 1), lambda i: (0, 0)),
            pl.BlockSpec((H, H), lambda i: (0, 0)),
            pl.BlockSpec((1, H), lambda i: (0, 0)),
        ],
        out_specs=[
            pl.BlockSpec((EB, 1), lambda i: (i, 0)),
            pl.BlockSpec((EB, H), lambda i: (i, 0)),
        ],
        out_shape=[
            jax.ShapeDtypeStruct((E, 1), jnp.float32),
            jax.ShapeDtypeStruct((E, H), jnp.float32),
        ],
    )(a2, o2, We, be.reshape(1, -1), Weo, beo.reshape(1, -1),
      _bf(Wl1), bl1.reshape(1, 1), _bf(Wl2), bl2.reshape(1, H))


def _t0h(a2, o2, We, be, Weo, beo, Wl, bl):
    return pl.pallas_call(
        _t0h_body,
        grid=(E // EB,),
        in_specs=_EE_SPECS + [
            pl.BlockSpec((H, H), lambda i: (0, 0)),
            pl.BlockSpec((1, H), lambda i: (0, 0)),
        ],
        out_specs=pl.BlockSpec((EB, H), lambda i: (i, 0)),
        out_shape=jax.ShapeDtypeStruct((E, H), jnp.float32),
    )(a2, o2, We, be.reshape(1, -1), Weo, beo.reshape(1, -1),
      _bf(Wl), bl.reshape(1, H))


@functools.partial(
    pl.kernel,
    out_type=jax.ShapeDtypeStruct((NC, NP), jnp.float32),
    mesh=_MESH,
    compiler_params=_SC_PARAMS,
    scratch_types=dict(
        xs_v=pltpu.VMEM((NP,), jnp.float32),
        src_v=pltpu.VMEM((EC,), jnp.int32),
        dst_v=pltpu.VMEM((EC,), jnp.int32),
        ea_v=pltpu.VMEM((EC,), jnp.float32),
        acc_v=pltpu.VMEM((NP,), jnp.float32),
        red_v=pltpu.VMEM((NS, NCHUNK), jnp.float32),
        out_v=pltpu.VMEM((NCHUNK,), jnp.float32),
        shared=pltpu.VMEM_SHARED((NS, NP), jnp.float32),
    ),
)
def _k1(xs_hbm, ei_hbm, ea_hbm, out_hbm,
        xs_v, src_v, dst_v, ea_v, acc_v, red_v, out_v, shared):
    cid = lax.axis_index("c")
    sid = lax.axis_index("s")
    wid = sid * NC + cid
    ebase = wid * EC

    pltpu.sync_copy(xs_hbm, xs_v)
    pltpu.sync_copy(ei_hbm.at[pl.ds(ebase, EC)], src_v)
    pltpu.sync_copy(ei_hbm.at[pl.ds(E + ebase, EC)], dst_v)
    pltpu.sync_copy(ea_hbm.at[pl.ds(ebase, EC)], ea_v)

    zeros = jnp.zeros((L,), jnp.float32)

    def _zero(i, _):
        acc_v[pl.ds(i * L, L)] = zeros
        return 0
    lax.fori_loop(0, NP // L, _zero, 0)

    def _edge_group(g, _):
        idx = src_v[pl.ds(g * L, L)]
        xg = plsc.load_gather(xs_v, [idx])
        m = jnp.maximum(xg + ea_v[pl.ds(g * L, L)], 0.0)
        d = dst_v[pl.ds(g * L, L)]
        plsc.addupdate_scatter(acc_v, [d], m)
        return 0
    lax.fori_loop(0, EC // L, _edge_group, 0)

    pltpu.sync_copy(acc_v, shared.at[sid])
    plsc.subcore_barrier()
    pltpu.sync_copy(shared.at[:, pl.ds(sid * NCHUNK, NCHUNK)], red_v)

    def _red(j, _):
        s = red_v[0, pl.ds(j * L, L)]
        for rrow in range(1, NS):
            s = s + red_v[rrow, pl.ds(j * L, L)]
        out_v[pl.ds(j * L, L)] = s
        return 0
    lax.fori_loop(0, NCHUNK // L, _red, 0)
    pltpu.sync_copy(out_v, out_hbm.at[cid, pl.ds(sid * NCHUNK, NCHUNK)])


@functools.partial(
    pl.kernel,
    out_type=jax.ShapeDtypeStruct((NC, NP, H), jnp.float32),
    mesh=_MESH,
    compiler_params=_SC_PARAMS,
    scratch_types=dict(
        src_v=pltpu.VMEM((EC,), jnp.int32),
        rows0=pltpu.VMEM((CHUNK, H), jnp.float32),
        rows1=pltpu.VMEM((CHUNK, H), jnp.float32),
        ea_v=pltpu.VMEM((CHUNK, H), jnp.float32),
        didx0=pltpu.VMEM((CHUNK,), jnp.int32),
        didx1=pltpu.VMEM((CHUNK,), jnp.int32),
        acc_s=pltpu.VMEM_SHARED((NP, H), jnp.float32),
        g_sem0=pltpu.SemaphoreType.DMA,
        g_sem1=pltpu.SemaphoreType.DMA,
        ea_sem=pltpu.SemaphoreType.DMA,
        d_sem0=pltpu.SemaphoreType.DMA,
        d_sem1=pltpu.SemaphoreType.DMA,
    ),
)
def _edge_pass(table_hbm, ei_hbm, ea_hbm, out_hbm,
               src_v, rows0, rows1, ea_v, didx0, didx1, acc_s,
               g_sem0, g_sem1, ea_sem, d_sem0, d_sem1):
    cid = lax.axis_index("c")
    sid = lax.axis_index("s")
    wid = sid * NC + cid
    ebase = wid * EC
    rbase = sid * NCHUNK
    rows = (rows0, rows1)
    didx = (didx0, didx1)
    g_sem = (g_sem0, g_sem1)
    d_sem = (d_sem0, d_sem1)

    pltpu.sync_copy(ei_hbm.at[pl.ds(ebase, EC)], src_v)

    zeros = jnp.zeros((L,), jnp.float32)

    def _zrow(i, _):
        for blk in range(H // L):
            rows0[i, pl.ds(blk * L, L)] = zeros
        return 0
    lax.fori_loop(0, CHUNK, _zrow, 0)
    for piece in range(NCHUNK // CHUNK):
        pltpu.sync_copy(rows0, acc_s.at[pl.ds(rbase + piece * CHUNK, CHUNK)])
    plsc.subcore_barrier()

    def _issue_gather(c, b):
        return pltpu.async_copy(
            table_hbm.at[src_v.at[pl.ds(c * CHUNK, CHUNK)]], rows[b], g_sem[b])

    def _issue_didx(c, b):
        return pltpu.async_copy(
            ei_hbm.at[pl.ds(E + ebase + c * CHUNK, CHUNK)], didx[b], d_sem[b])

    def _issue_ea(c):
        return pltpu.async_copy(
            ea_hbm.at[pl.ds(ebase + c * CHUNK, CHUNK)], ea_v, ea_sem)

    def _relu_chunk(b):
        def body(r, _):
            for blk in range(H // L):
                sl = pl.ds(blk * L, L)
                rows[b][r, sl] = jnp.maximum(rows[b][r, sl] + ea_v[r, sl], 0.0)
            return 0
        lax.fori_loop(0, CHUNK, body, 0)

    def _step(c, b):
        pltpu.make_async_copy(
            table_hbm.at[src_v.at[pl.ds(0, CHUNK)]], rows[b], g_sem[b]).wait()
        pltpu.make_async_copy(
            ea_hbm.at[pl.ds(0, CHUNK)], ea_v, ea_sem).wait()
        pltpu.make_async_copy(
            ei_hbm.at[pl.ds(0, CHUNK)], didx[b], d_sem[b]).wait()
        _relu_chunk(b)

        @pl.when(c + 1 < NCH)
        def _():
            _issue_ea(c + 1)
        pltpu.sync_copy(rows[b], acc_s.at[didx[b]], add=True)

        @pl.when(c + 2 < NCH)
        def _():
            _issue_gather(c + 2, b)
            _issue_didx(c + 2, b)

    _issue_gather(0, 0)
    _issue_didx(0, 0)
    _issue_gather(1, 1)
    _issue_didx(1, 1)
    _issue_ea(0)

    def _pair(i, _):
        _step(2 * i, 0)
        _step(2 * i + 1, 1)
        return 0
    lax.fori_loop(0, NCH // 2, _pair, 0)
    if NCH % 2:
        _step(NCH - 1, 0)
    plsc.subcore_barrier()

    for piece in range(NCHUNK // CHUNK):
        pbase = rbase + piece * CHUNK
        pltpu.sync_copy(acc_s.at[pl.ds(pbase, CHUNK)], rows0)
        pltpu.sync_copy(rows0, out_hbm.at[cid, pl.ds(pbase, CHUNK)])


def _k3a_body(x_ref, p0_ref, p1_ref, w_ref, b_ref, o_ref):
    g1 = x_ref[...] + p0_ref[...] + p1_ref[...]
    o_ref[...] = g1 * w_ref[...] + b_ref[...]


def _k3a(xs2, p02, p12, w1r, bn1):
    cspec = pl.BlockSpec((RB, 1), lambda i: (i, 0))
    return pl.pallas_call(
        _k3a_body,
        grid=(NP // RB,),
        in_specs=[cspec, cspec, cspec,
                  pl.BlockSpec((1, H), lambda i: (0, 0)),
                  pl.BlockSpec((1, H), lambda i: (0, 0))],
        out_specs=pl.BlockSpec((RB, H), lambda i: (i, 0)),
        out_shape=jax.ShapeDtypeStruct((NP, H), jnp.float32),
    )(xs2, p02, p12, w1r, bn1.reshape(1, H))


def _k3_body(h1_ref, a_ref, w_ref, b_ref, o_ref):
    t = _bf(h1_ref[...] + a_ref[0] + a_ref[1])
    o_ref[...] = jnp.dot(t, w_ref[...], preferred_element_type=jnp.float32) + b_ref[...]


def _k3(h1, A, Wn, bn):
    return pl.pallas_call(
        _k3_body,
        grid=(NP // RB,),
        in_specs=[
            pl.BlockSpec((RB, H), lambda i: (i, 0)),
            pl.BlockSpec((NC, RB, H), lambda i: (0, i, 0)),
            pl.BlockSpec((H, H), lambda i: (0, 0)),
            pl.BlockSpec((1, H), lambda i: (0, 0)),
        ],
        out_specs=pl.BlockSpec((RB, H), lambda i: (i, 0)),
        out_shape=jax.ShapeDtypeStruct((NP, H), jnp.float32),
    )(h1, A, _bf(Wn), bn.reshape(1, H))


def _k5_body(h2_ref, b_ref, w3_ref, b3_ref, bid_ref, wf1_ref, bf1_ref,
             wf2_ref, bf2_ref, o_ref, pool_ref):
    i = pl.program_id(0)

    @pl.when(i == 0)
    def _():
        pool_ref[...] = jnp.full((G, H), NEG, jnp.float32)

    t = _bf(h2_ref[...] + b_ref[0] + b_ref[1])
    h3 = jnp.dot(t, w3_ref[...], preferred_element_type=jnp.float32) + b3_ref[...]
    bid = bid_ref[...]
    bmin = bid_ref[0, 0]
    bmax = bid_ref[RB - 1, 0]
    for g in range(G):
        @pl.when((g >= bmin) & (g <= bmax))
        def _():
            masked = jnp.where(bid == g, h3, NEG)
            red = jnp.max(masked, axis=0)
            pool_ref[g, :] = jnp.maximum(pool_ref[g, :], red)

    @pl.when(i == NP // RB - 1)
    def _():
        pooled = pool_ref[...]
        z = jax.nn.relu(
            jnp.dot(_bf(pooled), wf1_ref[...], preferred_element_type=jnp.float32)
            + bf1_ref[...])
        o_ref[...] = (jnp.dot(_bf(z), wf2_ref[...], preferred_element_type=jnp.float32)
                      + bf2_ref[...])


def _k5(h2, B, Wn3, bn3, bid, Wf1, bf1, Wf2, bf2):
    return pl.pallas_call(
        _k5_body,
        grid=(NP // RB,),
        in_specs=[
            pl.BlockSpec((RB, H), lambda i: (i, 0)),
            pl.BlockSpec((NC, RB, H), lambda i: (0, i, 0)),
            pl.BlockSpec((H, H), lambda i: (0, 0)),
            pl.BlockSpec((1, H), lambda i: (0, 0)),
            pl.BlockSpec((RB, 1), lambda i: (i, 0)),
            pl.BlockSpec((H, H), lambda i: (0, 0)),
            pl.BlockSpec((1, H), lambda i: (0, 0)),
            pl.BlockSpec((H, 1), lambda i: (0, 0)),
            pl.BlockSpec((1, 1), lambda i: (0, 0)),
        ],
        out_specs=pl.BlockSpec((G, 1), lambda i: (0, 0)),
        out_shape=jax.ShapeDtypeStruct((G, 1), jnp.float32),
        scratch_shapes=[pltpu.VMEM((G, H), jnp.float32)],
    )(h2, B, _bf(Wn3), bn3.reshape(1, H), bid, _bf(Wf1), bf1.reshape(1, H),
      _bf(Wf2), bf2.reshape(1, 1))


def kernel(x, edge_index, edge_attr, batch, opt_edge, We, be, Weo, beo, Wl1, bl1, Wn1, bn1, Wl2, bl2, Wn2, bn2, Wl3, bl3, Wn3, bn3, Wf1, bf1, Wf2, bf2):
    xs = jnp.pad(x[:, 0], (0, NP - N))
    eif = edge_index.reshape(2 * E)

    ea1, ea2 = _t0a(edge_attr, opt_edge, We, be, Weo, beo, Wl1, bl1, Wl2, bl2)
    part1 = _k1(xs, eif, ea1.reshape(E))

    w1r = Wn1[0].reshape(1, H)
    h1 = _k3a(xs.reshape(NP, 1), part1[0].reshape(NP, 1),
              part1[1].reshape(NP, 1), w1r, bn1)

    A = _edge_pass(h1, eif, ea2)
    ea3 = _t0h(edge_attr, opt_edge, We, be, Weo, beo, Wl3, bl3)
    h2 = _k3(h1, A, Wn2, bn2)

    B = _edge_pass(h2, eif, ea3)

    bid = jnp.pad(batch, (0, NP - N), constant_values=G).reshape(NP, 1)
    return _k5(h2, B, Wn3, bn3, bid, Wf1, bf1, Wf2, bf2)
